# Initial kernel scaffold; baseline (speedup 1.0000x reference)
import jax
import jax.numpy as jnp
from jax.experimental import pallas as pl


def kernel(x):
    raise NotImplementedError("write your pallas kernel here")


# speedup vs baseline: 2.2192x; 2.2192x over previous
import jax
import jax.numpy as jnp
import numpy as np
from jax import lax
from jax.experimental import pallas as pl
from jax.experimental.pallas import tpu as pltpu
from jax.experimental.pallas import tpu_sc as plsc

NC = 2
NS = 16
NW = NC * NS
L = 16

ROWS = 128
N = 32768
V = N // L
S = 2
J = V // S
RB = 8
NB = 1 << RB
HW = L * NB
NPASS = 4
ROWS_PER_W = ROWS // NW

_MIN_I32 = np.int32(-2147483648)


def _flip(k):
    m = lax.shift_right_arithmetic(k, 31) | _MIN_I32
    return k ^ m


def _unflip(k):
    m = lax.shift_right_arithmetic(~k, 31) | _MIN_I32
    return k ^ m


def _sort_body(x_hbm, out_hbm, buf_a, buf_b, hist0, hist1, totals):
    wid = lax.axis_index("s") * NC + lax.axis_index("c")
    iota = lax.iota(jnp.int32, L)
    lane_base = lax.shift_left(iota, RB)
    zeros = iota & np.int32(0)
    ones = zeros + np.int32(1)
    hists = (hist0, hist1)
    mask_d = np.int32(NB - 1)

    def do_row(rr, _):
        row = wid * ROWS_PER_W + rr
        pltpu.sync_copy(x_hbm.at[row], buf_a)

        for p in range(NPASS):
            shift = RB * p
            first = p == 0
            last = p == NPASS - 1
            src = buf_a if p % 2 == 0 else buf_b
            dst = buf_b if p % 2 == 0 else buf_a

            def zero_body(c, _, hists=hists):
                for h in hists:
                    h[pl.ds(c * L, L)] = zeros
                return 0

            lax.fori_loop(0, HW // L, zero_body, 0)

            def p1_body(j, _, src=src, shift=shift, first=first, hists=hists):
                for s in range(S):
                    sl = pl.ds(j * (L * S) + s * L, L)
                    k = src[sl]
                    if first:
                        k = _flip(k)
                        src[sl] = k
                    d = k if shift == 0 else lax.shift_right_logical(k, shift)
                    d = d & mask_d
                    plsc.addupdate_scatter(hists[s], [lane_base + d], ones)
                return 0

            lax.fori_loop(0, J, p1_body, 0)

            def sum_body(c, _, hists=hists):
                acc = zeros
                for l in range(L):
                    for s in range(S):
                        acc = acc + hists[s][pl.ds(l * NB + c * L, L)]
                totals[pl.ds(c * L, L)] = acc
                return 0

            lax.fori_loop(0, NB // L, sum_body, 0)

            def scan_body(c, carry):
                v = totals[pl.ds(c * L, L)]
                cum = plsc.cumsum(v)
                totals[pl.ds(c * L, L)] = cum - v + carry
                return carry + jnp.sum(v)

            lax.fori_loop(0, NB // L, scan_body, np.int32(0))

            def col_body(c, _, hists=hists):
                acc = totals[pl.ds(c * L, L)]
                for l in range(L):
                    for s in range(S):
                        sl = pl.ds(l * NB + c * L, L)
                        v = hists[s][sl]
                        hists[s][sl] = acc
                        acc = acc + v
                return 0

            lax.fori_loop(0, NB // L, col_body, 0)

            def p3_body(j, _, src=src, dst=dst, shift=shift, last=last,
                        hists=hists):
                for s in range(S):
                    k = src[pl.ds(j * (L * S) + s * L, L)]
                    d = k if shift == 0 else lax.shift_right_logical(k, shift)
                    d = d & mask_d
                    idx = lane_base + d
                    r = plsc.load_gather(hists[s], [idx])
                    plsc.store_scatter(hists[s], [idx], r + 1)
                    if last:
                        q = r
                        k = _unflip(k)
                    else:
                        q = (
                            lax.shift_left(r & np.int32(J - 1), 5)
                            | lax.shift_left(
                                lax.shift_right_logical(r, 10) & np.int32(1), 4
                            )
                            | lax.shift_right_logical(r, 11)
                        )
                    plsc.store_scatter(dst, [q], k)
                return 0

            lax.fori_loop(0, J, p3_body, 0)

        pltpu.sync_copy(buf_a, out_hbm.at[row])
        return 0

    lax.fori_loop(0, ROWS_PER_W, do_row, 0)


_mesh = plsc.VectorSubcoreMesh(
    core_axis_name="c", subcore_axis_name="s", num_cores=NC, num_subcores=NS
)

_sort = pl.kernel(
    _sort_body,
    out_type=jax.ShapeDtypeStruct((ROWS, N), jnp.int32),
    mesh=_mesh,
    scratch_types=[
        pltpu.VMEM((N,), jnp.int32),
        pltpu.VMEM((N,), jnp.int32),
        pltpu.VMEM((HW,), jnp.int32),
        pltpu.VMEM((HW,), jnp.int32),
        pltpu.VMEM((NB,), jnp.int32),
    ],
    compiler_params=pltpu.CompilerParams(needs_layout_passes=False),
)


@jax.jit
def kernel(x):
    x_i32 = lax.bitcast_convert_type(x, jnp.int32)
    out = _sort(x_i32)
    return lax.bitcast_convert_type(out, jnp.float32)


# speedup vs baseline: 2.2530x; 1.0152x over previous
import jax
import jax.numpy as jnp
import numpy as np
from jax import lax
from jax.experimental import pallas as pl
from jax.experimental.pallas import tpu as pltpu
from jax.experimental.pallas import tpu_sc as plsc

NC = 2
NS = 16
NW = NC * NS
L = 16

ROWS = 128
N = 32768
V = N // L
S = 4
LOG_S = 2
J = V // S
LOG_J = 11 - LOG_S
U = 2
RB = 8
NB = 1 << RB
HW = L * NB
NPASS = 4
ROWS_PER_W = ROWS // NW

_MIN_I32 = np.int32(-2147483648)


def _flip(k):
    m = lax.shift_right_arithmetic(k, 31) | _MIN_I32
    return k ^ m


def _unflip(k):
    m = lax.shift_right_arithmetic(~k, 31) | _MIN_I32
    return k ^ m


def _sort_body(x_hbm, out_hbm, buf_a, buf_b, h0, h1, h2, h3, totals):
    wid = lax.axis_index("s") * NC + lax.axis_index("c")
    iota = lax.iota(jnp.int32, L)
    lane_base = lax.shift_left(iota, RB)
    zeros = iota & np.int32(0)
    ones = zeros + np.int32(1)
    hists = (h0, h1, h2, h3)[:S]
    mask_d = np.int32(NB - 1)

    def do_row(rr, _):
        row = wid * ROWS_PER_W + rr
        pltpu.sync_copy(x_hbm.at[row], buf_a)

        for p in range(NPASS):
            shift = RB * p
            first = p == 0
            last = p == NPASS - 1
            src = buf_a if p % 2 == 0 else buf_b
            dst = buf_b if p % 2 == 0 else buf_a

            def zero_body(c, _, hists=hists):
                for u in range(8):
                    for h in hists:
                        h[pl.ds(c * (8 * L) + u * L, L)] = zeros
                return 0

            lax.fori_loop(0, HW // (8 * L), zero_body, 0)

            def p1_body(j, _, src=src, shift=shift, first=first, hists=hists):
                for u in range(U):
                    for s in range(S):
                        sl = pl.ds(j * (L * S * U) + (u * S + s) * L, L)
                        k = src[sl]
                        if first:
                            k = _flip(k)
                            src[sl] = k
                        d = k if shift == 0 else lax.shift_right_logical(k, shift)
                        d = d & mask_d
                        plsc.addupdate_scatter(hists[s], [lane_base + d], ones)
                return 0

            lax.fori_loop(0, J // U, p1_body, 0)

            def sum_body(c, _, hists=hists):
                acc = zeros
                for l in range(L):
                    for s in range(S):
                        acc = acc + hists[s][pl.ds(l * NB + c * L, L)]
                totals[pl.ds(c * L, L)] = acc
                return 0

            lax.fori_loop(0, NB // L, sum_body, 0)

            def scan_body(c, carry):
                v = totals[pl.ds(c * L, L)]
                cum = plsc.cumsum(v)
                totals[pl.ds(c * L, L)] = cum - v + carry
                return carry + jnp.sum(v)

            lax.fori_loop(0, NB // L, scan_body, np.int32(0))

            def col_body(c, _, hists=hists):
                acc = totals[pl.ds(c * L, L)]
                for l in range(L):
                    for s in range(S):
                        sl = pl.ds(l * NB + c * L, L)
                        v = hists[s][sl]
                        hists[s][sl] = acc
                        acc = acc + v
                return 0

            lax.fori_loop(0, NB // L, col_body, 0)

            def p3_body(j, _, src=src, dst=dst, shift=shift, last=last,
                        hists=hists):
                for u in range(U):
                    for s in range(S):
                        k = src[pl.ds(j * (L * S * U) + (u * S + s) * L, L)]
                        d = k if shift == 0 else lax.shift_right_logical(k, shift)
                        d = d & mask_d
                        idx = lane_base + d
                        r = plsc.load_gather(hists[s], [idx])
                        plsc.store_scatter(hists[s], [idx], r + 1)
                        if last:
                            q = r
                            k = _unflip(k)
                        else:
                            q = (
                                lax.shift_left(r & np.int32(J - 1), 4 + LOG_S)
                                | lax.shift_left(
                                    lax.shift_right_logical(r, LOG_J)
                                    & np.int32(S - 1),
                                    4,
                                )
                                | lax.shift_right_logical(r, 11)
                            )
                        plsc.store_scatter(dst, [q], k)
                return 0

            lax.fori_loop(0, J // U, p3_body, 0)

        pltpu.sync_copy(buf_a, out_hbm.at[row])
        return 0

    lax.fori_loop(0, ROWS_PER_W, do_row, 0)


_mesh = plsc.VectorSubcoreMesh(
    core_axis_name="c", subcore_axis_name="s", num_cores=NC, num_subcores=NS
)

_sort = pl.kernel(
    _sort_body,
    out_type=jax.ShapeDtypeStruct((ROWS, N), jnp.int32),
    mesh=_mesh,
    scratch_types=[
        pltpu.VMEM((N,), jnp.int32),
        pltpu.VMEM((N,), jnp.int32),
        pltpu.VMEM((HW,), jnp.int32),
        pltpu.VMEM((HW,), jnp.int32),
        pltpu.VMEM((HW,), jnp.int32),
        pltpu.VMEM((HW,), jnp.int32),
        pltpu.VMEM((NB,), jnp.int32),
    ],
    compiler_params=pltpu.CompilerParams(needs_layout_passes=False),
)


@jax.jit
def kernel(x):
    x_i32 = lax.bitcast_convert_type(x, jnp.int32)
    out = _sort(x_i32)
    return lax.bitcast_convert_type(out, jnp.float32)


# speedup vs baseline: 5.8662x; 2.6038x over previous
import jax
import jax.numpy as jnp
import numpy as np
from jax import lax
from jax.experimental import pallas as pl
from jax.experimental.pallas import tpu as pltpu
from jax.experimental.pallas import tpu_sc as plsc

NC = 2
NS = 16
NW = NC * NS
L = 16

ROWS = 128
N = 32768
V = N // L
S = 8
LOG_S = 3
J = V // S
LOG_J = 11 - LOG_S
RB = 8
NB = 1 << RB
HW = L * NB
NPASS = 4
ROWS_PER_W = ROWS // NW

_MIN_I32 = np.int32(-2147483648)


def _flip(k):
    m = lax.shift_right_arithmetic(k, 31) | _MIN_I32
    return k ^ m


def _unflip(k):
    m = lax.shift_right_arithmetic(~k, 31) | _MIN_I32
    return k ^ m


def _tree_sum(vs):
    while len(vs) > 1:
        vs = [vs[i] + vs[i + 1] for i in range(0, len(vs) - 1, 2)] + (
            [vs[-1]] if len(vs) % 2 else []
        )
    return vs[0]


def _sort_body(x_hbm, out_hbm, buf_a, buf_b, h0, h1, h2, h3, h4, h5, h6, h7,
               totals):
    wid = lax.axis_index("s") * NC + lax.axis_index("c")
    iota = lax.iota(jnp.int32, L)
    lane_base = lax.shift_left(iota, RB)
    zeros = iota & np.int32(0)
    ones = zeros + np.int32(1)
    hists = (h0, h1, h2, h3, h4, h5, h6, h7)[:S]
    mask_d = np.int32(NB - 1)

    def digit(k, shift):
        d = k if shift == 0 else lax.shift_right_logical(k, shift)
        return d & mask_d

    def do_row(rr, _):
        row = wid * ROWS_PER_W + rr
        pltpu.sync_copy(x_hbm.at[row], buf_a)

        for p in range(NPASS):
            shift = RB * p
            first = p == 0
            last = p == NPASS - 1
            src = buf_a if p % 2 == 0 else buf_b
            dst = buf_b if p % 2 == 0 else buf_a

            def zero_body(c, _, hists=hists):
                for h in hists:
                    h[pl.ds(c * L, L)] = zeros
                return 0

            lax.fori_loop(0, HW // L, zero_body, 0)

            def p1_body(j, _, src=src, shift=shift, first=first, hists=hists):
                base = j * (L * S)
                ks = [src[pl.ds(base + s * L, L)] for s in range(S)]
                if first:
                    ks = [_flip(k) for k in ks]
                    for s in range(S):
                        src[pl.ds(base + s * L, L)] = ks[s]
                idxs = [lane_base + digit(k, shift) for k in ks]
                for s in range(S):
                    plsc.addupdate_scatter(hists[s], [idxs[s]], ones)
                return 0

            lax.fori_loop(0, J, p1_body, 0)

            def sum_body(c, _, hists=hists):
                acc = zeros
                for l in range(L):
                    vs = [
                        hists[s][pl.ds(l * NB + c * L, L)] for s in range(S)
                    ]
                    acc = acc + _tree_sum(vs)
                totals[pl.ds(c * L, L)] = acc
                return 0

            lax.fori_loop(0, NB // L, sum_body, 0)

            def scan_body(c, carry):
                v = totals[pl.ds(c * L, L)]
                cum = plsc.cumsum(v)
                totals[pl.ds(c * L, L)] = cum - v + carry
                return carry + jnp.sum(v)

            lax.fori_loop(0, NB // L, scan_body, np.int32(0))

            def col_body(c, _, hists=hists):
                acc = totals[pl.ds(c * L, L)]
                for l in range(L):
                    vs = [
                        hists[s][pl.ds(l * NB + c * L, L)] for s in range(S)
                    ]
                    for s in range(S):
                        hists[s][pl.ds(l * NB + c * L, L)] = acc
                        acc = acc + vs[s]
                return 0

            lax.fori_loop(0, NB // L, col_body, 0)

            def p3_body(j, _, src=src, dst=dst, shift=shift, last=last,
                        hists=hists):
                base = j * (L * S)
                ks = [src[pl.ds(base + s * L, L)] for s in range(S)]
                idxs = [lane_base + digit(k, shift) for k in ks]
                rs = [
                    plsc.load_gather(hists[s], [idxs[s]]) for s in range(S)
                ]
                for s in range(S):
                    plsc.store_scatter(hists[s], [idxs[s]], rs[s] + ones)
                if last:
                    qs = rs
                    outs = [_unflip(k) for k in ks]
                else:
                    qs = [
                        lax.shift_left(r & np.int32(J - 1), 4 + LOG_S)
                        | lax.shift_left(
                            lax.shift_right_logical(r, LOG_J)
                            & np.int32(S - 1),
                            4,
                        )
                        | lax.shift_right_logical(r, 11)
                        for r in rs
                    ]
                    outs = ks
                for s in range(S):
                    plsc.store_scatter(dst, [qs[s]], outs[s])
                return 0

            lax.fori_loop(0, J, p3_body, 0)

        pltpu.sync_copy(buf_a, out_hbm.at[row])
        return 0

    lax.fori_loop(0, ROWS_PER_W, do_row, 0)


_mesh = plsc.VectorSubcoreMesh(
    core_axis_name="c", subcore_axis_name="s", num_cores=NC, num_subcores=NS
)

_sort = pl.kernel(
    _sort_body,
    out_type=jax.ShapeDtypeStruct((ROWS, N), jnp.int32),
    mesh=_mesh,
    scratch_types=[
        pltpu.VMEM((N,), jnp.int32),
        pltpu.VMEM((N,), jnp.int32),
        pltpu.VMEM((HW,), jnp.int32),
        pltpu.VMEM((HW,), jnp.int32),
        pltpu.VMEM((HW,), jnp.int32),
        pltpu.VMEM((HW,), jnp.int32),
        pltpu.VMEM((HW,), jnp.int32),
        pltpu.VMEM((HW,), jnp.int32),
        pltpu.VMEM((HW,), jnp.int32),
        pltpu.VMEM((HW,), jnp.int32),
        pltpu.VMEM((NB,), jnp.int32),
    ],
    compiler_params=pltpu.CompilerParams(needs_layout_passes=False),
)


@jax.jit
def kernel(x):
    x_i32 = lax.bitcast_convert_type(x, jnp.int32)
    out = _sort(x_i32)
    return lax.bitcast_convert_type(out, jnp.float32)


# speedup vs baseline: 6.0602x; 1.0331x over previous
import jax
import jax.numpy as jnp
import numpy as np
from jax import lax
from jax.experimental import pallas as pl
from jax.experimental.pallas import tpu as pltpu
from jax.experimental.pallas import tpu_sc as plsc

NC = 2
NS = 16
NW = NC * NS
L = 16

ROWS = 128
N = 32768
V = N // L
S = 8
LOG_S = 3
J = V // S
LOG_J = 11 - LOG_S
RB = 8
NB = 1 << RB
HW = L * NB
NPASS = 4
ROWS_PER_W = ROWS // NW

_MIN_I32 = np.int32(-2147483648)


def _flip(k):
    m = lax.shift_right_arithmetic(k, 31) | _MIN_I32
    return k ^ m


def _unflip(k):
    m = lax.shift_right_arithmetic(~k, 31) | _MIN_I32
    return k ^ m


def _tree_sum(vs):
    while len(vs) > 1:
        vs = [vs[i] + vs[i + 1] for i in range(0, len(vs) - 1, 2)] + (
            [vs[-1]] if len(vs) % 2 else []
        )
    return vs[0]


def _sort_body(x_hbm, out_hbm, buf_a, buf_b, h0, h1, h2, h3, h4, h5, h6, h7,
               totals):
    wid = lax.axis_index("s") * NC + lax.axis_index("c")
    iota = lax.iota(jnp.int32, L)
    lane_base = lax.shift_left(iota, RB)
    zeros = iota & np.int32(0)
    ones = zeros + np.int32(1)
    hists = (h0, h1, h2, h3, h4, h5, h6, h7)[:S]
    mask_d = np.int32(NB - 1)

    def digit(k, shift):
        d = k if shift == 0 else lax.shift_right_logical(k, shift)
        return d & mask_d

    def do_row(rr, _):
        row = wid * ROWS_PER_W + rr
        pltpu.sync_copy(x_hbm.at[row], buf_a)

        for p in range(NPASS):
            shift = RB * p
            first = p == 0
            last = p == NPASS - 1
            src = buf_a if p % 2 == 0 else buf_b
            dst = buf_b if p % 2 == 0 else buf_a

            def zero_body(c, _, hists=hists):
                for h in hists:
                    h[pl.ds(c * L, L)] = zeros
                return 0

            lax.fori_loop(0, HW // L, zero_body, 0)

            def p1_body(j, _, src=src, shift=shift, first=first, hists=hists):
                base = j * L
                ks = [src[pl.ds(base + s * (J * L), L)] for s in range(S)]
                if first:
                    ks = [_flip(k) for k in ks]
                    for s in range(S):
                        src[pl.ds(base + s * (J * L), L)] = ks[s]
                idxs = [lane_base + digit(k, shift) for k in ks]
                for s in range(S):
                    plsc.addupdate_scatter(hists[s], [idxs[s]], ones)
                return 0

            lax.fori_loop(0, J, p1_body, 0)

            def sum_body(c, _, hists=hists):
                acc = zeros
                for l in range(L):
                    vs = [
                        hists[s][pl.ds(l * NB + c * L, L)] for s in range(S)
                    ]
                    acc = acc + _tree_sum(vs)
                totals[pl.ds(c * L, L)] = acc
                return 0

            lax.fori_loop(0, NB // L, sum_body, 0)

            def scan_body(c, carry):
                v = totals[pl.ds(c * L, L)]
                cum = plsc.cumsum(v)
                totals[pl.ds(c * L, L)] = cum - v + carry
                return carry + jnp.sum(v)

            lax.fori_loop(0, NB // L, scan_body, np.int32(0))

            def col_body(c, _, hists=hists):
                acc = totals[pl.ds(c * L, L)]
                for l in range(L):
                    vs = [
                        hists[s][pl.ds(l * NB + c * L, L)] for s in range(S)
                    ]
                    for s in range(S):
                        hists[s][pl.ds(l * NB + c * L, L)] = lax.shift_left(
                            acc, 4
                        )
                        acc = acc + vs[s]
                return 0

            lax.fori_loop(0, NB // L, col_body, 0)

            def p3_body(j, _, src=src, dst=dst, shift=shift, last=last,
                        hists=hists):
                base = j * L
                ks = [src[pl.ds(base + s * (J * L), L)] for s in range(S)]
                idxs = [lane_base + digit(k, shift) for k in ks]
                rs = [
                    plsc.load_gather(hists[s], [idxs[s]]) for s in range(S)
                ]
                sixteens = lax.shift_left(ones, 4)
                for s in range(S):
                    plsc.store_scatter(hists[s], [idxs[s]], rs[s] + sixteens)
                if last:
                    qs = [lax.shift_right_logical(r, 4) for r in rs]
                    outs = [_unflip(k) for k in ks]
                else:
                    qs = [
                        (r & np.int32(0x7FF0))
                        | lax.shift_right_logical(r, 15)
                        for r in rs
                    ]
                    outs = ks
                for s in range(S):
                    plsc.store_scatter(dst, [qs[s]], outs[s])
                return 0

            lax.fori_loop(0, J, p3_body, 0)

        pltpu.sync_copy(buf_a, out_hbm.at[row])
        return 0

    lax.fori_loop(0, ROWS_PER_W, do_row, 0)


_mesh = plsc.VectorSubcoreMesh(
    core_axis_name="c", subcore_axis_name="s", num_cores=NC, num_subcores=NS
)

_sort = pl.kernel(
    _sort_body,
    out_type=jax.ShapeDtypeStruct((ROWS, N), jnp.int32),
    mesh=_mesh,
    scratch_types=[
        pltpu.VMEM((N,), jnp.int32),
        pltpu.VMEM((N,), jnp.int32),
        pltpu.VMEM((HW,), jnp.int32),
        pltpu.VMEM((HW,), jnp.int32),
        pltpu.VMEM((HW,), jnp.int32),
        pltpu.VMEM((HW,), jnp.int32),
        pltpu.VMEM((HW,), jnp.int32),
        pltpu.VMEM((HW,), jnp.int32),
        pltpu.VMEM((HW,), jnp.int32),
        pltpu.VMEM((HW,), jnp.int32),
        pltpu.VMEM((NB,), jnp.int32),
    ],
    compiler_params=pltpu.CompilerParams(needs_layout_passes=False),
)


@jax.jit
def kernel(x):
    x_i32 = lax.bitcast_convert_type(x, jnp.int32)
    out = _sort(x_i32)
    return lax.bitcast_convert_type(out, jnp.float32)


# speedup vs baseline: 6.5280x; 1.0772x over previous
import jax
import jax.numpy as jnp
import numpy as np
from jax import lax
from jax.experimental import pallas as pl
from jax.experimental.pallas import tpu as pltpu
from jax.experimental.pallas import tpu_sc as plsc

NC = 2
NS = 16
NW = NC * NS
L = 16

ROWS = 128
N = 32768
V = N // L
S = 4
U = 2
J = V // (S * U)
RB = 8
NB = 1 << RB
HW = L * NB
NPASS = 4
ROWS_PER_W = ROWS // NW
SB = J * U * L

_MIN_I32 = np.int32(-2147483648)


def _flip(k):
    m = lax.shift_right_arithmetic(k, 31) | _MIN_I32
    return k ^ m


def _unflip(k):
    m = lax.shift_right_arithmetic(~k, 31) | _MIN_I32
    return k ^ m


def _tree_sum(vs):
    while len(vs) > 1:
        vs = [vs[i] + vs[i + 1] for i in range(0, len(vs) - 1, 2)] + (
            [vs[-1]] if len(vs) % 2 else []
        )
    return vs[0]


def _sort_body(x_hbm, out_hbm, buf_a, buf_b, h0, h1, h2, h3, totals):
    wid = lax.axis_index("s") * NC + lax.axis_index("c")
    iota = lax.iota(jnp.int32, L)
    lane_base = lax.shift_left(iota, RB)
    zeros = iota & np.int32(0)
    ones = zeros + np.int32(1)
    sixteens = lax.shift_left(ones, 4)
    hists = (h0, h1, h2, h3)[:S]
    mask_d = np.int32(NB - 1)

    def digit(k, shift):
        d = k if shift == 0 else lax.shift_right_logical(k, shift)
        return d & mask_d

    def do_row(rr, _):
        row = wid * ROWS_PER_W + rr
        pltpu.sync_copy(x_hbm.at[row], buf_a)

        for p in range(NPASS):
            shift = RB * p
            first = p == 0
            last = p == NPASS - 1
            src = buf_a if p % 2 == 0 else buf_b
            dst = buf_b if p % 2 == 0 else buf_a

            def zero_body(c, _, hists=hists):
                for u in range(2):
                    for h in hists:
                        h[pl.ds(c * (2 * L) + u * L, L)] = zeros
                return 0

            lax.fori_loop(0, HW // (2 * L), zero_body, 0)

            def p1_body(j, _, src=src, shift=shift, first=first, hists=hists):
                sls = [
                    pl.ds(s * SB + j * (U * L) + u * L, L)
                    for u in range(U)
                    for s in range(S)
                ]
                ks = [src[sl] for sl in sls]
                if first:
                    ks = [_flip(k) for k in ks]
                    for sl, k in zip(sls, ks):
                        src[sl] = k
                idxs = [lane_base + digit(k, shift) for k in ks]
                for i in range(U * S):
                    plsc.addupdate_scatter(hists[i % S], [idxs[i]], ones)
                return 0

            lax.fori_loop(0, J, p1_body, 0)

            def sum_body(c, _, hists=hists):
                acc = zeros
                for l in range(L):
                    vs = [
                        hists[s][pl.ds(l * NB + c * L, L)] for s in range(S)
                    ]
                    acc = acc + _tree_sum(vs)
                totals[pl.ds(c * L, L)] = acc
                return 0

            lax.fori_loop(0, NB // L, sum_body, 0)

            def scan_body(c, carry):
                v = totals[pl.ds(c * L, L)]
                cum = plsc.cumsum(v)
                totals[pl.ds(c * L, L)] = cum - v + carry
                return carry + jnp.sum(v)

            lax.fori_loop(0, NB // L, scan_body, np.int32(0))

            def col_body(c, _, hists=hists):
                acc = totals[pl.ds(c * L, L)]
                for l in range(L):
                    vs = [
                        hists[s][pl.ds(l * NB + c * L, L)] for s in range(S)
                    ]
                    for s in range(S):
                        hists[s][pl.ds(l * NB + c * L, L)] = lax.shift_left(
                            acc, 4
                        )
                        acc = acc + vs[s]
                return 0

            lax.fori_loop(0, NB // L, col_body, 0)

            def p3_body(j, _, src=src, dst=dst, shift=shift, last=last,
                        hists=hists):
                sls = [
                    pl.ds(s * SB + j * (U * L) + u * L, L)
                    for u in range(U)
                    for s in range(S)
                ]
                ks = [src[sl] for sl in sls]
                idxs = [lane_base + digit(k, shift) for k in ks]
                rs = []
                for u in range(U):
                    rs_u = [
                        plsc.load_gather(hists[s], [idxs[u * S + s]])
                        for s in range(S)
                    ]
                    for s in range(S):
                        plsc.store_scatter(
                            hists[s], [idxs[u * S + s]], rs_u[s] + sixteens
                        )
                    rs.extend(rs_u)
                if last:
                    qs = [lax.shift_right_logical(r, 4) for r in rs]
                    outs = [_unflip(k) for k in ks]
                else:
                    qs = [
                        (r & np.int32(0x7FF0))
                        | lax.shift_right_logical(r, 15)
                        for r in rs
                    ]
                    outs = ks
                for i in range(U * S):
                    plsc.store_scatter(dst, [qs[i]], outs[i])
                return 0

            lax.fori_loop(0, J, p3_body, 0)

        pltpu.sync_copy(buf_a, out_hbm.at[row])
        return 0

    lax.fori_loop(0, ROWS_PER_W, do_row, 0)


_mesh = plsc.VectorSubcoreMesh(
    core_axis_name="c", subcore_axis_name="s", num_cores=NC, num_subcores=NS
)

_sort = pl.kernel(
    _sort_body,
    out_type=jax.ShapeDtypeStruct((ROWS, N), jnp.int32),
    mesh=_mesh,
    scratch_types=[
        pltpu.VMEM((N,), jnp.int32),
        pltpu.VMEM((N,), jnp.int32),
        pltpu.VMEM((HW,), jnp.int32),
        pltpu.VMEM((HW,), jnp.int32),
        pltpu.VMEM((HW,), jnp.int32),
        pltpu.VMEM((HW,), jnp.int32),
        pltpu.VMEM((NB,), jnp.int32),
    ],
    compiler_params=pltpu.CompilerParams(needs_layout_passes=False),
)


@jax.jit
def kernel(x):
    x_i32 = lax.bitcast_convert_type(x, jnp.int32)
    out = _sort(x_i32)
    return lax.bitcast_convert_type(out, jnp.float32)


# speedup vs baseline: 6.6818x; 1.0236x over previous
import jax
import jax.numpy as jnp
import numpy as np
from jax import lax
from jax.experimental import pallas as pl
from jax.experimental.pallas import tpu as pltpu
from jax.experimental.pallas import tpu_sc as plsc

NC = 2
NS = 16
NW = NC * NS
L = 16

ROWS = 128
N = 32768
V = N // L
S = 4
U = 2
J = V // (S * U)
RB = 8
NB = 1 << RB
HW = L * NB
NPASS = 4
ROWS_PER_W = ROWS // NW
SB = J * U * L

_MIN_I32 = np.int32(-2147483648)


def _flip(k):
    m = lax.shift_right_arithmetic(k, 31) | _MIN_I32
    return k ^ m


def _unflip(k):
    m = lax.shift_right_arithmetic(~k, 31) | _MIN_I32
    return k ^ m


def _tree_sum(vs):
    while len(vs) > 1:
        vs = [vs[i] + vs[i + 1] for i in range(0, len(vs) - 1, 2)] + (
            [vs[-1]] if len(vs) % 2 else []
        )
    return vs[0]


def _sort_body(x_hbm, out_hbm, buf_a, buf_b, buf_c, h0, h1, h2, h3, totals,
               in_sem, out_sem):
    wid = lax.axis_index("s") * NC + lax.axis_index("c")
    iota = lax.iota(jnp.int32, L)
    lane_base = lax.shift_left(iota, RB)
    zeros = iota & np.int32(0)
    ones = zeros + np.int32(1)
    sixteens = lax.shift_left(ones, 4)
    hists = (h0, h1, h2, h3)[:S]
    mask_d = np.int32(NB - 1)
    base_row = wid * ROWS_PER_W

    def digit(k, shift):
        d = k if shift == 0 else lax.shift_right_logical(k, shift)
        return d & mask_d

    def wait_in():
        pltpu.make_async_copy(x_hbm.at[0], buf_b, in_sem).wait()

    def wait_out():
        pltpu.make_async_copy(x_hbm.at[0], buf_b, out_sem).wait()

    def sort_row(home, row, hook):
        for p in range(NPASS):
            shift = RB * p
            first = p == 0
            last = p == NPASS - 1
            src = home if p % 2 == 0 else buf_b
            dst = buf_b if p % 2 == 0 else home

            def zero_body(c, _, hists=hists):
                for u in range(2):
                    for h in hists:
                        h[pl.ds(c * (2 * L) + u * L, L)] = zeros
                return 0

            lax.fori_loop(0, HW // (2 * L), zero_body, 0)

            def p1_body(j, _, src=src, shift=shift, first=first, hists=hists):
                sls = [
                    pl.ds(s * SB + j * (U * L) + u * L, L)
                    for u in range(U)
                    for s in range(S)
                ]
                ks = [src[sl] for sl in sls]
                if first:
                    ks = [_flip(k) for k in ks]
                    for sl, k in zip(sls, ks):
                        src[sl] = k
                idxs = [lane_base + digit(k, shift) for k in ks]
                for i in range(U * S):
                    plsc.addupdate_scatter(hists[i % S], [idxs[i]], ones)
                return 0

            lax.fori_loop(0, J, p1_body, 0)

            if first:
                hook()

            def sum_body(c, _, hists=hists):
                acc = zeros
                for l in range(L):
                    vs = [
                        hists[s][pl.ds(l * NB + c * L, L)] for s in range(S)
                    ]
                    acc = acc + _tree_sum(vs)
                totals[pl.ds(c * L, L)] = acc
                return 0

            lax.fori_loop(0, NB // L, sum_body, 0)

            def scan_body(c, carry):
                v = totals[pl.ds(c * L, L)]
                cum = plsc.cumsum(v)
                totals[pl.ds(c * L, L)] = cum - v + carry
                return carry + jnp.sum(v)

            lax.fori_loop(0, NB // L, scan_body, np.int32(0))

            def col_body(c, _, hists=hists):
                acc = totals[pl.ds(c * L, L)]
                for l in range(L):
                    vs = [
                        hists[s][pl.ds(l * NB + c * L, L)] for s in range(S)
                    ]
                    for s in range(S):
                        hists[s][pl.ds(l * NB + c * L, L)] = lax.shift_left(
                            acc, 4
                        )
                        acc = acc + vs[s]
                return 0

            lax.fori_loop(0, NB // L, col_body, 0)

            def p3_body(j, _, src=src, dst=dst, shift=shift, last=last,
                        hists=hists):
                sls = [
                    pl.ds(s * SB + j * (U * L) + u * L, L)
                    for u in range(U)
                    for s in range(S)
                ]
                ks = [src[sl] for sl in sls]
                idxs = [lane_base + digit(k, shift) for k in ks]
                rs = []
                for u in range(U):
                    rs_u = [
                        plsc.load_gather(hists[s], [idxs[u * S + s]])
                        for s in range(S)
                    ]
                    for s in range(S):
                        plsc.store_scatter(
                            hists[s], [idxs[u * S + s]], rs_u[s] + sixteens
                        )
                    rs.extend(rs_u)
                if last:
                    qs = [lax.shift_right_logical(r, 4) for r in rs]
                    outs = [_unflip(k) for k in ks]
                else:
                    qs = [
                        (r & np.int32(0x7FF0))
                        | lax.shift_right_logical(r, 15)
                        for r in rs
                    ]
                    outs = ks
                for i in range(U * S):
                    plsc.store_scatter(dst, [qs[i]], outs[i])
                return 0

            lax.fori_loop(0, J, p3_body, 0)

    def pair_body(k, _):
        row_a = base_row + 2 * k
        row_b = row_a + 1

        def hook_a(k=k, row_b=row_b):
            @pl.when(k > 0)
            def _():
                wait_out()

            pltpu.async_copy(x_hbm.at[row_b], buf_c, in_sem)

        @pl.when(k > 0)
        def _():
            wait_in()

        sort_row(buf_a, row_a, hook_a)
        pltpu.async_copy(buf_a, out_hbm.at[row_a], out_sem)
        wait_in()

        def hook_b(k=k, row_a=row_a):
            wait_out()

            @pl.when(k == 0)
            def _():
                pltpu.async_copy(x_hbm.at[row_a + 2], buf_a, in_sem)

        sort_row(buf_c, row_b, hook_b)
        pltpu.async_copy(buf_c, out_hbm.at[row_b], out_sem)
        return 0

    pltpu.sync_copy(x_hbm.at[base_row], buf_a)
    lax.fori_loop(0, ROWS_PER_W // 2, pair_body, 0)
    wait_out()


_mesh = plsc.VectorSubcoreMesh(
    core_axis_name="c", subcore_axis_name="s", num_cores=NC, num_subcores=NS
)

_sort = pl.kernel(
    _sort_body,
    out_type=jax.ShapeDtypeStruct((ROWS, N), jnp.int32),
    mesh=_mesh,
    scratch_types=[
        pltpu.VMEM((N,), jnp.int32),
        pltpu.VMEM((N,), jnp.int32),
        pltpu.VMEM((N,), jnp.int32),
        pltpu.VMEM((HW,), jnp.int32),
        pltpu.VMEM((HW,), jnp.int32),
        pltpu.VMEM((HW,), jnp.int32),
        pltpu.VMEM((HW,), jnp.int32),
        pltpu.VMEM((NB,), jnp.int32),
        pltpu.SemaphoreType.DMA,
        pltpu.SemaphoreType.DMA,
    ],
    compiler_params=pltpu.CompilerParams(needs_layout_passes=False),
)


@jax.jit
def kernel(x):
    x_i32 = lax.bitcast_convert_type(x, jnp.int32)
    out = _sort(x_i32)
    return lax.bitcast_convert_type(out, jnp.float32)


# speedup vs baseline: 7.9981x; 1.1970x over previous
import jax
import jax.numpy as jnp
import numpy as np
from jax import lax
from jax.experimental import pallas as pl
from jax.experimental.pallas import tpu as pltpu
from jax.experimental.pallas import tpu_sc as plsc

NC = 2
NS = 16
NW = NC * NS
L = 16

ROWS = 128
N = 32768
V = N // L
S = 8
J = V // S
RB = 8
NB = 1 << RB
NPASS = 4
ROWS_PER_W = ROWS // NW
SB = J * L

_MIN_I32 = np.int32(-2147483648)


def _flip(k):
    m = lax.shift_right_arithmetic(k, 31) | _MIN_I32
    return k ^ m


def _unflip(k):
    m = lax.shift_right_arithmetic(~k, 31) | _MIN_I32
    return k ^ m


def _tree_sum(vs):
    while len(vs) > 1:
        vs = [vs[i] + vs[i + 1] for i in range(0, len(vs) - 1, 2)] + (
            [vs[-1]] if len(vs) % 2 else []
        )
    return vs[0]


def _sort_body(x_hbm, out_hbm, buf_a, buf_b, buf_c, h0, h1, h2, h3, h4, h5,
               h6, h7, totals, in_sem, out_sem):
    wid = lax.axis_index("s") * NC + lax.axis_index("c")
    iota = lax.iota(jnp.int32, L)
    zeros = iota & np.int32(0)
    ones = zeros + np.int32(1)
    hists = (h0, h1, h2, h3, h4, h5, h6, h7)[:S]
    mask_d = np.int32(NB - 1)
    base_row = wid * ROWS_PER_W

    def digit(k, shift):
        d = k if shift == 0 else lax.shift_right_logical(k, shift)
        return d & mask_d

    def wait_in():
        pltpu.make_async_copy(x_hbm.at[0], buf_b, in_sem).wait()

    def wait_out():
        pltpu.make_async_copy(x_hbm.at[0], buf_b, out_sem).wait()

    def sort_row(home, row, hook):
        for p in range(NPASS):
            shift = RB * p
            first = p == 0
            last = p == NPASS - 1
            src = home if p % 2 == 0 else buf_b
            dst = buf_b if p % 2 == 0 else home

            def zero_body(c, _, hists=hists):
                for h in hists:
                    h[pl.ds(c * L, L)] = zeros
                return 0

            lax.fori_loop(0, NB // L, zero_body, 0)

            def p1_body(j, _, src=src, shift=shift, first=first, hists=hists):
                sls = [pl.ds(s * SB + j * L, L) for s in range(S)]
                ks = [src[sl] for sl in sls]
                if first:
                    ks = [_flip(k) for k in ks]
                    for sl, k in zip(sls, ks):
                        src[sl] = k
                ds_ = [digit(k, shift) for k in ks]
                ccs = [plsc.scan_count(d) for d in ds_]
                for s in range(S):
                    occ, last_m = ccs[s]
                    plsc.addupdate_scatter(
                        hists[s], [ds_[s]], occ, mask=last_m
                    )
                return 0

            lax.fori_loop(0, J, p1_body, 0)

            if first:
                hook()

            def sum_body(c, _, hists=hists):
                vs = [hists[s][pl.ds(c * L, L)] for s in range(S)]
                totals[pl.ds(c * L, L)] = _tree_sum(vs)
                return 0

            lax.fori_loop(0, NB // L, sum_body, 0)

            def scan_body(c, carry):
                v = totals[pl.ds(c * L, L)]
                cum = plsc.cumsum(v)
                totals[pl.ds(c * L, L)] = cum - v + carry
                return carry + jnp.sum(v)

            lax.fori_loop(0, NB // L, scan_body, np.int32(0))

            def col_body(c, _, hists=hists):
                acc = totals[pl.ds(c * L, L)]
                vs = [hists[s][pl.ds(c * L, L)] for s in range(S)]
                for s in range(S):
                    hists[s][pl.ds(c * L, L)] = acc
                    acc = acc + vs[s]
                return 0

            lax.fori_loop(0, NB // L, col_body, 0)

            def p3_body(j, _, src=src, dst=dst, shift=shift, last=last,
                        hists=hists):
                sls = [pl.ds(s * SB + j * L, L) for s in range(S)]
                ks = [src[sl] for sl in sls]
                ds_ = [digit(k, shift) for k in ks]
                ccs = [plsc.scan_count(d) for d in ds_]
                rbs = [
                    plsc.load_gather(hists[s], [ds_[s]]) for s in range(S)
                ]
                qs = []
                for s in range(S):
                    occ, last_m = ccs[s]
                    val = rbs[s] + occ
                    plsc.store_scatter(
                        hists[s], [ds_[s]], val, mask=last_m
                    )
                    qs.append(val - ones)
                outs = [_unflip(k) for k in ks] if last else ks
                for s in range(S):
                    plsc.store_scatter(dst, [qs[s]], outs[s])
                return 0

            lax.fori_loop(0, J, p3_body, 0)

    def pair_body(k, _):
        row_a = base_row + 2 * k
        row_b = row_a + 1

        def hook_a(k=k, row_b=row_b):
            @pl.when(k > 0)
            def _():
                wait_out()

            pltpu.async_copy(x_hbm.at[row_b], buf_c, in_sem)

        @pl.when(k > 0)
        def _():
            wait_in()

        sort_row(buf_a, row_a, hook_a)
        pltpu.async_copy(buf_a, out_hbm.at[row_a], out_sem)
        wait_in()

        def hook_b(k=k, row_a=row_a):
            wait_out()

            @pl.when(k == 0)
            def _():
                pltpu.async_copy(x_hbm.at[row_a + 2], buf_a, in_sem)

        sort_row(buf_c, row_b, hook_b)
        pltpu.async_copy(buf_c, out_hbm.at[row_b], out_sem)
        return 0

    pltpu.sync_copy(x_hbm.at[base_row], buf_a)
    lax.fori_loop(0, ROWS_PER_W // 2, pair_body, 0)
    wait_out()


_mesh = plsc.VectorSubcoreMesh(
    core_axis_name="c", subcore_axis_name="s", num_cores=NC, num_subcores=NS
)

_sort = pl.kernel(
    _sort_body,
    out_type=jax.ShapeDtypeStruct((ROWS, N), jnp.int32),
    mesh=_mesh,
    scratch_types=[
        pltpu.VMEM((N,), jnp.int32),
        pltpu.VMEM((N,), jnp.int32),
        pltpu.VMEM((N,), jnp.int32),
        pltpu.VMEM((NB,), jnp.int32),
        pltpu.VMEM((NB,), jnp.int32),
        pltpu.VMEM((NB,), jnp.int32),
        pltpu.VMEM((NB,), jnp.int32),
        pltpu.VMEM((NB,), jnp.int32),
        pltpu.VMEM((NB,), jnp.int32),
        pltpu.VMEM((NB,), jnp.int32),
        pltpu.VMEM((NB,), jnp.int32),
        pltpu.VMEM((NB,), jnp.int32),
        pltpu.SemaphoreType.DMA,
        pltpu.SemaphoreType.DMA,
    ],
    compiler_params=pltpu.CompilerParams(needs_layout_passes=False),
)


@jax.jit
def kernel(x):
    x_i32 = lax.bitcast_convert_type(x, jnp.int32)
    out = _sort(x_i32)
    return lax.bitcast_convert_type(out, jnp.float32)


# speedup vs baseline: 8.5004x; 1.0628x over previous
import jax
import jax.numpy as jnp
import numpy as np
from jax import lax
from jax.experimental import pallas as pl
from jax.experimental.pallas import tpu as pltpu
from jax.experimental.pallas import tpu_sc as plsc

NC = 2
NS = 16
NW = NC * NS
L = 16

ROWS = 128
N = 32768
V = N // L
S = 8
J = V // S
RB = 8
NB = 1 << RB
HW = S * NB
NPASS = 4
ROWS_PER_W = ROWS // NW
SB = J * L
LOG_SB = 12

_MIN_I32 = np.int32(-2147483648)


def _flip(k):
    m = lax.shift_right_arithmetic(k, 31) | _MIN_I32
    return k ^ m


def _unflip(k):
    m = lax.shift_right_arithmetic(~k, 31) | _MIN_I32
    return k ^ m


def _tree_sum(vs):
    while len(vs) > 1:
        vs = [vs[i] + vs[i + 1] for i in range(0, len(vs) - 1, 2)] + (
            [vs[-1]] if len(vs) % 2 else []
        )
    return vs[0]


def _sort_body(x_hbm, out_hbm, buf_a, buf_b, buf_c, hist0, hist1, totals,
               in_sem, out_sem):
    wid = lax.axis_index("s") * NC + lax.axis_index("c")
    iota = lax.iota(jnp.int32, L)
    zeros = iota & np.int32(0)
    ones = zeros + np.int32(1)
    mask_d = np.int32(NB - 1)
    base_row = wid * ROWS_PER_W

    def digit(k, shift):
        d = k if shift == 0 else lax.shift_right_logical(k, shift)
        return d & mask_d

    def wait_in():
        pltpu.make_async_copy(x_hbm.at[0], buf_b, in_sem).wait()

    def wait_out():
        pltpu.make_async_copy(x_hbm.at[0], buf_b, out_sem).wait()

    def zero_hist(h):
        def zero_body(c, _):
            for u in range(8):
                h[pl.ds(c * (8 * L) + u * L, L)] = zeros
            return 0

        lax.fori_loop(0, HW // (8 * L), zero_body, 0)

    def sort_row(home, row, hook):
        for p in range(NPASS):
            shift = RB * p
            first = p == 0
            last = p == NPASS - 1
            src = home if p % 2 == 0 else buf_b
            dst = buf_b if p % 2 == 0 else home
            cur = hist0 if p % 2 == 0 else hist1
            nxt = hist1 if p % 2 == 0 else hist0
            cur_views = [cur.at[pl.ds(s * NB, NB)] for s in range(S)]

            if first:
                zero_hist(cur)

                def p1_body(j, _, src=src):
                    sls = [pl.ds(s * SB + j * L, L) for s in range(S)]
                    ks = [_flip(src[sl]) for sl in sls]
                    for sl, k in zip(sls, ks):
                        src[sl] = k
                    ds_ = [digit(k, 0) for k in ks]
                    ccs = [plsc.scan_count(d) for d in ds_]
                    for s in range(S):
                        occ, last_m = ccs[s]
                        plsc.addupdate_scatter(
                            cur_views[s], [ds_[s]], occ, mask=last_m
                        )
                    return 0

                lax.fori_loop(0, J, p1_body, 0)
                hook()

            def sum_body(c, _, cur=cur):
                vs = [cur[pl.ds(s * NB + c * L, L)] for s in range(S)]
                totals[pl.ds(c * L, L)] = _tree_sum(vs)
                return 0

            lax.fori_loop(0, NB // L, sum_body, 0)

            def scan_body(c, carry):
                v = totals[pl.ds(c * L, L)]
                cum = plsc.cumsum(v)
                totals[pl.ds(c * L, L)] = cum - v + carry
                return carry + jnp.sum(v)

            lax.fori_loop(0, NB // L, scan_body, np.int32(0))

            def col_body(c, _, cur=cur):
                acc = totals[pl.ds(c * L, L)]
                vs = [cur[pl.ds(s * NB + c * L, L)] for s in range(S)]
                for s in range(S):
                    cur[pl.ds(s * NB + c * L, L)] = acc
                    acc = acc + vs[s]
                return 0

            lax.fori_loop(0, NB // L, col_body, 0)

            if not last:
                zero_hist(nxt)

            def p3_body(j, _, src=src, dst=dst, shift=shift, last=last,
                        cur_views=cur_views, nxt=nxt):
                sls = [pl.ds(s * SB + j * L, L) for s in range(S)]
                ks = [src[sl] for sl in sls]
                ds_ = [digit(k, shift) for k in ks]
                ccs = [plsc.scan_count(d) for d in ds_]
                rbs = [
                    plsc.load_gather(cur_views[s], [ds_[s]])
                    for s in range(S)
                ]
                qs = []
                for s in range(S):
                    occ, last_m = ccs[s]
                    val = rbs[s] + occ
                    plsc.store_scatter(
                        cur_views[s], [ds_[s]], val, mask=last_m
                    )
                    qs.append(val - ones)
                if last:
                    outs = [_unflip(k) for k in ks]
                else:
                    outs = ks
                    nis = [
                        lax.shift_left(
                            lax.shift_right_logical(q, LOG_SB), RB
                        )
                        | digit(k, shift + RB)
                        for q, k in zip(qs, ks)
                    ]
                    nccs = [plsc.scan_count(ni) for ni in nis]
                for s in range(S):
                    plsc.store_scatter(dst, [qs[s]], outs[s])
                if not last:
                    for s in range(S):
                        nocc, nlast = nccs[s]
                        plsc.addupdate_scatter(
                            nxt, [nis[s]], nocc, mask=nlast
                        )
                return 0

            lax.fori_loop(0, J, p3_body, 0)

    def pair_body(k, _):
        row_a = base_row + 2 * k
        row_b = row_a + 1

        def hook_a(k=k, row_b=row_b):
            @pl.when(k > 0)
            def _():
                wait_out()

            pltpu.async_copy(x_hbm.at[row_b], buf_c, in_sem)

        @pl.when(k > 0)
        def _():
            wait_in()

        sort_row(buf_a, row_a, hook_a)
        pltpu.async_copy(buf_a, out_hbm.at[row_a], out_sem)
        wait_in()

        def hook_b(k=k, row_a=row_a):
            wait_out()

            @pl.when(k == 0)
            def _():
                pltpu.async_copy(x_hbm.at[row_a + 2], buf_a, in_sem)

        sort_row(buf_c, row_b, hook_b)
        pltpu.async_copy(buf_c, out_hbm.at[row_b], out_sem)
        return 0

    pltpu.sync_copy(x_hbm.at[base_row], buf_a)
    lax.fori_loop(0, ROWS_PER_W // 2, pair_body, 0)
    wait_out()


_mesh = plsc.VectorSubcoreMesh(
    core_axis_name="c", subcore_axis_name="s", num_cores=NC, num_subcores=NS
)

_sort = pl.kernel(
    _sort_body,
    out_type=jax.ShapeDtypeStruct((ROWS, N), jnp.int32),
    mesh=_mesh,
    scratch_types=[
        pltpu.VMEM((N,), jnp.int32),
        pltpu.VMEM((N,), jnp.int32),
        pltpu.VMEM((N,), jnp.int32),
        pltpu.VMEM((HW,), jnp.int32),
        pltpu.VMEM((HW,), jnp.int32),
        pltpu.VMEM((NB,), jnp.int32),
        pltpu.SemaphoreType.DMA,
        pltpu.SemaphoreType.DMA,
    ],
    compiler_params=pltpu.CompilerParams(needs_layout_passes=False),
)


@jax.jit
def kernel(x):
    x_i32 = lax.bitcast_convert_type(x, jnp.int32)
    out = _sort(x_i32)
    return lax.bitcast_convert_type(out, jnp.float32)


# speedup vs baseline: 10.6185x; 1.2492x over previous
import jax
import jax.numpy as jnp
import numpy as np
from jax import lax
from jax.experimental import pallas as pl
from jax.experimental.pallas import tpu as pltpu
from jax.experimental.pallas import tpu_sc as plsc

NC = 2
NS = 16
NW = NC * NS
L = 16

ROWS = 128
N = 32768
V = N // L
S = 8
J = V // S
RB = 8
NB = 1 << RB
HW = S * NB
NPASS = 4
ROWS_PER_W = ROWS // NW
SB = J * L
LOG_SB = 12

_MIN_I32 = np.int32(-2147483648)


def _flip(k):
    m = lax.shift_right_arithmetic(k, 31) | _MIN_I32
    return k ^ m


def _unflip(k):
    m = lax.shift_right_arithmetic(~k, 31) | _MIN_I32
    return k ^ m


def _tree_sum(vs):
    while len(vs) > 1:
        vs = [vs[i] + vs[i + 1] for i in range(0, len(vs) - 1, 2)] + (
            [vs[-1]] if len(vs) % 2 else []
        )
    return vs[0]


def _sort_body(x_hbm, out_hbm, buf_a, buf_b, buf_c, hist0, hist1, totals,
               in_sem, out_sem):
    wid = lax.axis_index("s") * NC + lax.axis_index("c")
    iota = lax.iota(jnp.int32, L)
    zeros = iota & np.int32(0)
    ones = zeros + np.int32(1)
    mask_d = np.int32(NB - 1)
    base_row = wid * ROWS_PER_W

    def digit(k, shift):
        d = k if shift == 0 else lax.shift_right_logical(k, shift)
        return d & mask_d

    def wait_in():
        pltpu.make_async_copy(x_hbm.at[0], buf_b, in_sem).wait()

    def wait_out():
        pltpu.make_async_copy(x_hbm.at[0], buf_b, out_sem).wait()

    def zero_hist(h):
        def zero_body(c, _):
            for u in range(8):
                h[pl.ds(c * (8 * L) + u * L, L)] = zeros
            return 0

        lax.fori_loop(0, HW // (8 * L), zero_body, 0)

    def sort_row(home, row, hook):
        for p in range(NPASS):
            shift = RB * p
            first = p == 0
            last = p == NPASS - 1
            src = home if p % 2 == 0 else buf_b
            dst = buf_b if p % 2 == 0 else home
            cur = hist0 if p % 2 == 0 else hist1
            nxt = hist1 if p % 2 == 0 else hist0
            cur_views = [cur.at[pl.ds(s * NB, NB)] for s in range(S)]

            if first:
                zero_hist(cur)

                def p1_count(ks):
                    ds_ = [digit(k, 0) for k in ks]
                    ccs = [plsc.scan_count(d) for d in ds_]
                    return ds_, ccs

                def p1_add(ds_, ccs):
                    for s in range(S):
                        occ, last_m = ccs[s]
                        plsc.addupdate_scatter(
                            cur_views[s], [ds_[s]], occ, mask=last_m
                        )

                def p1_body(j, ks, src=src):
                    ds_, ccs = p1_count(ks)
                    nsls = [
                        pl.ds(s * SB + (j + 1) * L, L) for s in range(S)
                    ]
                    nks = [_flip(src[sl]) for sl in nsls]
                    for sl, k in zip(nsls, nks):
                        src[sl] = k
                    p1_add(ds_, ccs)
                    return nks

                ks0 = [_flip(src[pl.ds(s * SB, L)]) for s in range(S)]
                for s in range(S):
                    src[pl.ds(s * SB, L)] = ks0[s]
                ksl = lax.fori_loop(0, J - 1, p1_body, ks0)
                p1_add(*p1_count(ksl))
                hook()

            def sum_body(c, _, cur=cur):
                vs = [cur[pl.ds(s * NB + c * L, L)] for s in range(S)]
                totals[pl.ds(c * L, L)] = _tree_sum(vs)
                return 0

            lax.fori_loop(0, NB // L, sum_body, 0)

            def scan_body(c, carry):
                v = totals[pl.ds(c * L, L)]
                cum = plsc.cumsum(v)
                totals[pl.ds(c * L, L)] = cum - v + carry
                return carry + jnp.sum(v)

            lax.fori_loop(0, NB // L, scan_body, np.int32(0))

            def col_body(c, _, cur=cur):
                acc = totals[pl.ds(c * L, L)]
                vs = [cur[pl.ds(s * NB + c * L, L)] for s in range(S)]
                for s in range(S):
                    cur[pl.ds(s * NB + c * L, L)] = acc
                    acc = acc + vs[s]
                return 0

            lax.fori_loop(0, NB // L, col_body, 0)

            if not last:
                zero_hist(nxt)

            def p3_body(j, ks, src=src, dst=dst, shift=shift, last=last,
                        cur_views=cur_views, nxt=nxt):
                ds_ = [digit(k, shift) for k in ks]
                ccs = [plsc.scan_count(d) for d in ds_]
                jn = (j + 1) & np.int32(J - 1)
                nks = [src[pl.ds(s * SB + jn * L, L)] for s in range(S)]
                rbs = [
                    plsc.load_gather(cur_views[s], [ds_[s]])
                    for s in range(S)
                ]
                qs = []
                for s in range(S):
                    occ, last_m = ccs[s]
                    val = rbs[s] + occ
                    plsc.store_scatter(
                        cur_views[s], [ds_[s]], val, mask=last_m
                    )
                    qs.append(val - ones)
                if last:
                    outs = [_unflip(k) for k in ks]
                else:
                    outs = ks
                    nis = [
                        lax.shift_left(
                            lax.shift_right_logical(q, LOG_SB), RB
                        )
                        | digit(k, shift + RB)
                        for q, k in zip(qs, ks)
                    ]
                    nccs = [plsc.scan_count(ni) for ni in nis]
                for s in range(S):
                    plsc.store_scatter(dst, [qs[s]], outs[s])
                if not last:
                    for s in range(S):
                        nocc, nlast = nccs[s]
                        plsc.addupdate_scatter(
                            nxt, [nis[s]], nocc, mask=nlast
                        )
                return nks

            ks0 = [src[pl.ds(s * SB, L)] for s in range(S)]
            lax.fori_loop(0, J, p3_body, ks0)

    def pair_body(k, _):
        row_a = base_row + 2 * k
        row_b = row_a + 1

        def hook_a(k=k, row_b=row_b):
            @pl.when(k > 0)
            def _():
                wait_out()

            pltpu.async_copy(x_hbm.at[row_b], buf_c, in_sem)

        @pl.when(k > 0)
        def _():
            wait_in()

        sort_row(buf_a, row_a, hook_a)
        pltpu.async_copy(buf_a, out_hbm.at[row_a], out_sem)
        wait_in()

        def hook_b(k=k, row_a=row_a):
            wait_out()

            @pl.when(k == 0)
            def _():
                pltpu.async_copy(x_hbm.at[row_a + 2], buf_a, in_sem)

        sort_row(buf_c, row_b, hook_b)
        pltpu.async_copy(buf_c, out_hbm.at[row_b], out_sem)
        return 0

    pltpu.sync_copy(x_hbm.at[base_row], buf_a)
    lax.fori_loop(0, ROWS_PER_W // 2, pair_body, 0)
    wait_out()


_mesh = plsc.VectorSubcoreMesh(
    core_axis_name="c", subcore_axis_name="s", num_cores=NC, num_subcores=NS
)

_sort = pl.kernel(
    _sort_body,
    out_type=jax.ShapeDtypeStruct((ROWS, N), jnp.int32),
    mesh=_mesh,
    scratch_types=[
        pltpu.VMEM((N,), jnp.int32),
        pltpu.VMEM((N,), jnp.int32),
        pltpu.VMEM((N,), jnp.int32),
        pltpu.VMEM((HW,), jnp.int32),
        pltpu.VMEM((HW,), jnp.int32),
        pltpu.VMEM((NB,), jnp.int32),
        pltpu.SemaphoreType.DMA,
        pltpu.SemaphoreType.DMA,
    ],
    compiler_params=pltpu.CompilerParams(needs_layout_passes=False),
)


@jax.jit
def kernel(x):
    x_i32 = lax.bitcast_convert_type(x, jnp.int32)
    out = _sort(x_i32)
    return lax.bitcast_convert_type(out, jnp.float32)


# speedup vs baseline: 10.6188x; 1.0000x over previous
import jax
import jax.numpy as jnp
import numpy as np
from jax import lax
from jax.experimental import pallas as pl
from jax.experimental.pallas import tpu as pltpu
from jax.experimental.pallas import tpu_sc as plsc

NC = 2
NS = 16
NW = NC * NS
L = 16

ROWS = 128
N = 32768
V = N // L
S = 8
J = V // S
RB = 8
NB = 1 << RB
HW = S * NB
NPASS = 4
ROWS_PER_W = ROWS // NW
SB = J * L
LOG_SB = 12

_MIN_I32 = np.int32(-2147483648)


def _flip(k):
    m = lax.shift_right_arithmetic(k, 31) | _MIN_I32
    return k ^ m


def _unflip(k):
    m = lax.shift_right_arithmetic(~k, 31) | _MIN_I32
    return k ^ m


def _tree_sum(vs):
    while len(vs) > 1:
        vs = [vs[i] + vs[i + 1] for i in range(0, len(vs) - 1, 2)] + (
            [vs[-1]] if len(vs) % 2 else []
        )
    return vs[0]


def _sort_body(x_hbm, out_hbm, buf_a, buf_b, buf_c, hist0, hist1, totals,
               in_sem, out_sem):
    wid = lax.axis_index("s") * NC + lax.axis_index("c")
    iota = lax.iota(jnp.int32, L)
    zeros = iota & np.int32(0)
    ones = zeros + np.int32(1)
    mask_d = np.int32(NB - 1)
    base_row = wid * ROWS_PER_W

    def digit(k, shift):
        d = k if shift == 0 else lax.shift_right_logical(k, shift)
        return d & mask_d

    def wait_in():
        pltpu.make_async_copy(x_hbm.at[0], buf_b, in_sem).wait()

    def wait_out():
        pltpu.make_async_copy(x_hbm.at[0], buf_b, out_sem).wait()

    def zero_hist(h):
        def zero_body(c, _):
            for u in range(8):
                h[pl.ds(c * (8 * L) + u * L, L)] = zeros
            return 0

        lax.fori_loop(0, HW // (8 * L), zero_body, 0)

    def sort_row(home, row, hook):
        for p in range(NPASS):
            shift = RB * p
            first = p == 0
            last = p == NPASS - 1
            src = home if p % 2 == 0 else buf_b
            dst = buf_b if p % 2 == 0 else home
            cur = hist0 if p % 2 == 0 else hist1
            nxt = hist1 if p % 2 == 0 else hist0
            cur_views = [cur.at[pl.ds(s * NB, NB)] for s in range(S)]

            if first:
                zero_hist(cur)

                def p1_count(ks):
                    ds_ = [digit(k, 0) for k in ks]
                    ccs = [plsc.scan_count(d) for d in ds_]
                    return ds_, ccs

                def p1_add(ds_, ccs):
                    for s in range(S):
                        occ, last_m = ccs[s]
                        plsc.addupdate_scatter(
                            cur_views[s], [ds_[s]], occ, mask=last_m
                        )

                def p1_body(j, ks, src=src):
                    ds_, ccs = p1_count(ks)
                    nsls = [
                        pl.ds(s * SB + (j + 1) * L, L) for s in range(S)
                    ]
                    nks = [_flip(src[sl]) for sl in nsls]
                    for sl, k in zip(nsls, nks):
                        src[sl] = k
                    p1_add(ds_, ccs)
                    return nks

                ks0 = [_flip(src[pl.ds(s * SB, L)]) for s in range(S)]
                for s in range(S):
                    src[pl.ds(s * SB, L)] = ks0[s]
                ksl = lax.fori_loop(0, J - 1, p1_body, ks0)
                p1_add(*p1_count(ksl))
                hook()

            def sum_body(c, _, cur=cur):
                vs = [cur[pl.ds(s * NB + c * L, L)] for s in range(S)]
                totals[pl.ds(c * L, L)] = _tree_sum(vs)
                return 0

            lax.fori_loop(0, NB // L, sum_body, 0)

            def scan_body(c, carry):
                v = totals[pl.ds(c * L, L)]
                cum = plsc.cumsum(v)
                totals[pl.ds(c * L, L)] = cum - v + carry
                return carry + jnp.sum(v)

            lax.fori_loop(0, NB // L, scan_body, np.int32(0))

            def col_body(c, _, cur=cur):
                acc = totals[pl.ds(c * L, L)]
                vs = [cur[pl.ds(s * NB + c * L, L)] for s in range(S)]
                for s in range(S):
                    cur[pl.ds(s * NB + c * L, L)] = acc
                    acc = acc + vs[s]
                return 0

            lax.fori_loop(0, NB // L, col_body, 0)

            if not last:
                zero_hist(nxt)

            def p3_body(j, ks, src=src, dst=dst, shift=shift, last=last,
                        cur_views=cur_views, nxt=nxt):
                ds_ = [digit(k, shift) for k in ks]
                ccs = [plsc.scan_count(d) for d in ds_]
                jn = (j + 1) & np.int32(J - 1)
                nks = [src[pl.ds(s * SB + jn * L, L)] for s in range(S)]
                rbs = [
                    plsc.load_gather(cur_views[s], [ds_[s]])
                    for s in range(S)
                ]
                qs = []
                for s in range(S):
                    occ, last_m = ccs[s]
                    val = rbs[s] + occ
                    plsc.store_scatter(
                        cur_views[s], [ds_[s]], val, mask=last_m
                    )
                    qs.append(val + np.int32(-1))
                if last:
                    outs = [_unflip(k) for k in ks]
                else:
                    outs = ks
                    nis = [
                        (
                            lax.shift_right_logical(q, LOG_SB - RB)
                            & np.int32(((S - 1) << RB))
                        )
                        | digit(k, shift + RB)
                        for q, k in zip(qs, ks)
                    ]
                    nccs = [plsc.scan_count(ni) for ni in nis]
                for s in range(S):
                    plsc.store_scatter(dst, [qs[s]], outs[s])
                if not last:
                    for s in range(S):
                        nocc, nlast = nccs[s]
                        plsc.addupdate_scatter(
                            nxt, [nis[s]], nocc, mask=nlast
                        )
                return nks

            ks0 = [src[pl.ds(s * SB, L)] for s in range(S)]
            lax.fori_loop(0, J, p3_body, ks0)

    def pair_body(k, _):
        row_a = base_row + 2 * k
        row_b = row_a + 1

        def hook_a(k=k, row_b=row_b):
            @pl.when(k > 0)
            def _():
                wait_out()

            pltpu.async_copy(x_hbm.at[row_b], buf_c, in_sem)

        @pl.when(k > 0)
        def _():
            wait_in()

        sort_row(buf_a, row_a, hook_a)
        pltpu.async_copy(buf_a, out_hbm.at[row_a], out_sem)
        wait_in()

        def hook_b(k=k, row_a=row_a):
            wait_out()

            @pl.when(k == 0)
            def _():
                pltpu.async_copy(x_hbm.at[row_a + 2], buf_a, in_sem)

        sort_row(buf_c, row_b, hook_b)
        pltpu.async_copy(buf_c, out_hbm.at[row_b], out_sem)
        return 0

    pltpu.sync_copy(x_hbm.at[base_row], buf_a)
    lax.fori_loop(0, ROWS_PER_W // 2, pair_body, 0)
    wait_out()


_mesh = plsc.VectorSubcoreMesh(
    core_axis_name="c", subcore_axis_name="s", num_cores=NC, num_subcores=NS
)

_sort = pl.kernel(
    _sort_body,
    out_type=jax.ShapeDtypeStruct((ROWS, N), jnp.int32),
    mesh=_mesh,
    scratch_types=[
        pltpu.VMEM((N,), jnp.int32),
        pltpu.VMEM((N,), jnp.int32),
        pltpu.VMEM((N,), jnp.int32),
        pltpu.VMEM((HW,), jnp.int32),
        pltpu.VMEM((HW,), jnp.int32),
        pltpu.VMEM((NB,), jnp.int32),
        pltpu.SemaphoreType.DMA,
        pltpu.SemaphoreType.DMA,
    ],
    compiler_params=pltpu.CompilerParams(needs_layout_passes=False),
)


@jax.jit
def kernel(x):
    x_i32 = lax.bitcast_convert_type(x, jnp.int32)
    out = _sort(x_i32)
    return lax.bitcast_convert_type(out, jnp.float32)


# speedup vs baseline: 10.6621x; 1.0041x over previous
import jax
import jax.numpy as jnp
import numpy as np
from jax import lax
from jax.experimental import pallas as pl
from jax.experimental.pallas import tpu as pltpu
from jax.experimental.pallas import tpu_sc as plsc

NC = 2
NS = 16
NW = NC * NS
L = 16

ROWS = 128
N = 32768
V = N // L
S = 8
J = V // S
RB = 8
NB = 1 << RB
HW = S * NB
NPASS = 4
ROWS_PER_W = ROWS // NW
SB = J * L
LOG_SB = 12

_MIN_I32 = np.int32(-2147483648)


def _flip(k):
    m = lax.shift_right_arithmetic(k, 31) | _MIN_I32
    return k ^ m


def _unflip(k):
    m = lax.shift_right_arithmetic(~k, 31) | _MIN_I32
    return k ^ m


def _tree_sum(vs):
    while len(vs) > 1:
        vs = [vs[i] + vs[i + 1] for i in range(0, len(vs) - 1, 2)] + (
            [vs[-1]] if len(vs) % 2 else []
        )
    return vs[0]


def _sort_body(x_hbm, out_hbm, buf_a, buf_b, buf_c, hist0, hist1, totals,
               in_sem, out_sem):
    wid = lax.axis_index("s") * NC + lax.axis_index("c")
    iota = lax.iota(jnp.int32, L)
    zeros = iota & np.int32(0)
    ones = zeros + np.int32(1)
    mask_d = np.int32(NB - 1)
    base_row = wid * ROWS_PER_W

    def digit(k, shift):
        d = k if shift == 0 else lax.shift_right_logical(k, shift)
        return d & mask_d

    def wait_in():
        pltpu.make_async_copy(x_hbm.at[0], buf_b, in_sem).wait()

    def wait_out():
        pltpu.make_async_copy(x_hbm.at[0], buf_b, out_sem).wait()

    def zero_hist(h):
        def zero_body(c, _):
            for u in range(8):
                h[pl.ds(c * (8 * L) + u * L, L)] = zeros
            return 0

        lax.fori_loop(0, HW // (8 * L), zero_body, 0)

    def sort_row(home, row, hook):
        for p in range(NPASS):
            shift = RB * p
            first = p == 0
            last = p == NPASS - 1
            src = home if p % 2 == 0 else buf_b
            dst = buf_b if p % 2 == 0 else home
            cur = hist0 if p % 2 == 0 else hist1
            nxt = hist1 if p % 2 == 0 else hist0
            cur_views = [cur.at[pl.ds(s * NB, NB)] for s in range(S)]

            if first:
                zero_hist(cur)

                def p1_count(ks):
                    ds_ = [digit(k, 0) for k in ks]
                    ccs = [plsc.scan_count(d) for d in ds_]
                    return ds_, ccs

                def p1_add(ds_, ccs):
                    for s in range(S):
                        occ, last_m = ccs[s]
                        plsc.addupdate_scatter(
                            cur_views[s], [ds_[s]], occ, mask=last_m
                        )

                def p1_body(j, ks, src=src):
                    ds_, ccs = p1_count(ks)
                    nsls = [
                        pl.ds(s * SB + (j + 1) * L, L) for s in range(S)
                    ]
                    nks = [_flip(src[sl]) for sl in nsls]
                    for sl, k in zip(nsls, nks):
                        src[sl] = k
                    p1_add(ds_, ccs)
                    return nks

                ks0 = [_flip(src[pl.ds(s * SB, L)]) for s in range(S)]
                for s in range(S):
                    src[pl.ds(s * SB, L)] = ks0[s]
                ksl = lax.fori_loop(0, J - 1, p1_body, ks0)
                p1_add(*p1_count(ksl))
                hook()

            def sum_body(c, _, cur=cur):
                vs = [cur[pl.ds(s * NB + c * L, L)] for s in range(S)]
                totals[pl.ds(c * L, L)] = _tree_sum(vs)
                return 0

            lax.fori_loop(0, NB // L, sum_body, 0)

            def scan_body(c, carry):
                v = totals[pl.ds(c * L, L)]
                cum = plsc.cumsum(v)
                totals[pl.ds(c * L, L)] = cum - v + carry
                return carry + jnp.sum(v)

            lax.fori_loop(0, NB // L, scan_body, np.int32(0))

            def col_body(c, _, cur=cur, nxt=nxt, last=last):
                acc = totals[pl.ds(c * L, L)]
                vs = [cur[pl.ds(s * NB + c * L, L)] for s in range(S)]
                for s in range(S):
                    cur[pl.ds(s * NB + c * L, L)] = acc
                    if not last:
                        nxt[pl.ds(s * NB + c * L, L)] = zeros
                    acc = acc + vs[s]
                return 0

            lax.fori_loop(0, NB // L, col_body, 0)

            def p3_body(j, ks, src=src, dst=dst, shift=shift, last=last,
                        cur_views=cur_views, nxt=nxt):
                ds_ = [digit(k, shift) for k in ks]
                ccs = [plsc.scan_count(d) for d in ds_]
                jn = (j + 1) & np.int32(J - 1)
                nks = [src[pl.ds(s * SB + jn * L, L)] for s in range(S)]
                rbs = [
                    plsc.load_gather(cur_views[s], [ds_[s]])
                    for s in range(S)
                ]
                qs = []
                for s in range(S):
                    occ, last_m = ccs[s]
                    val = rbs[s] + occ
                    plsc.store_scatter(
                        cur_views[s], [ds_[s]], val, mask=last_m
                    )
                    qs.append(val + np.int32(-1))
                if last:
                    outs = [_unflip(k) for k in ks]
                else:
                    outs = ks
                    nis = [
                        (
                            lax.shift_right_logical(q, LOG_SB - RB)
                            & np.int32(((S - 1) << RB))
                        )
                        | digit(k, shift + RB)
                        for q, k in zip(qs, ks)
                    ]
                    nccs = [plsc.scan_count(ni) for ni in nis]
                for s in range(S):
                    plsc.store_scatter(dst, [qs[s]], outs[s])
                if not last:
                    for s in range(S):
                        nocc, nlast = nccs[s]
                        plsc.addupdate_scatter(
                            nxt, [nis[s]], nocc, mask=nlast
                        )
                return nks

            ks0 = [src[pl.ds(s * SB, L)] for s in range(S)]
            lax.fori_loop(0, J, p3_body, ks0)

    def pair_body(k, _):
        row_a = base_row + 2 * k
        row_b = row_a + 1

        def hook_a(k=k, row_b=row_b):
            @pl.when(k > 0)
            def _():
                wait_out()

            pltpu.async_copy(x_hbm.at[row_b], buf_c, in_sem)

        @pl.when(k > 0)
        def _():
            wait_in()

        sort_row(buf_a, row_a, hook_a)
        pltpu.async_copy(buf_a, out_hbm.at[row_a], out_sem)
        wait_in()

        def hook_b(k=k, row_a=row_a):
            wait_out()

            @pl.when(k == 0)
            def _():
                pltpu.async_copy(x_hbm.at[row_a + 2], buf_a, in_sem)

        sort_row(buf_c, row_b, hook_b)
        pltpu.async_copy(buf_c, out_hbm.at[row_b], out_sem)
        return 0

    pltpu.sync_copy(x_hbm.at[base_row], buf_a)
    lax.fori_loop(0, ROWS_PER_W // 2, pair_body, 0)
    wait_out()


_mesh = plsc.VectorSubcoreMesh(
    core_axis_name="c", subcore_axis_name="s", num_cores=NC, num_subcores=NS
)

_sort = pl.kernel(
    _sort_body,
    out_type=jax.ShapeDtypeStruct((ROWS, N), jnp.int32),
    mesh=_mesh,
    scratch_types=[
        pltpu.VMEM((N,), jnp.int32),
        pltpu.VMEM((N,), jnp.int32),
        pltpu.VMEM((N,), jnp.int32),
        pltpu.VMEM((HW,), jnp.int32),
        pltpu.VMEM((HW,), jnp.int32),
        pltpu.VMEM((NB,), jnp.int32),
        pltpu.SemaphoreType.DMA,
        pltpu.SemaphoreType.DMA,
    ],
    compiler_params=pltpu.CompilerParams(needs_layout_passes=False),
)


@jax.jit
def kernel(x):
    x_i32 = lax.bitcast_convert_type(x, jnp.int32)
    out = _sort(x_i32)
    return lax.bitcast_convert_type(out, jnp.float32)


# speedup vs baseline: 10.8066x; 1.0136x over previous
import jax
import jax.numpy as jnp
import numpy as np
from jax import lax
from jax.experimental import pallas as pl
from jax.experimental.pallas import tpu as pltpu
from jax.experimental.pallas import tpu_sc as plsc

NC = 2
NS = 16
NW = NC * NS
L = 16

ROWS = 128
N = 32768
V = N // L
S = 8
J = V // S
RB = 8
NB = 1 << RB
HW = S * NB
NPASS = 4
ROWS_PER_W = ROWS // NW
SB = J * L
LOG_SB = 12

_MIN_I32 = np.int32(-2147483648)


def _flip(k):
    m = lax.shift_right_arithmetic(k, 31) | _MIN_I32
    return k ^ m


def _unflip(k):
    m = lax.shift_right_arithmetic(~k, 31) | _MIN_I32
    return k ^ m


def _tree_sum(vs):
    while len(vs) > 1:
        vs = [vs[i] + vs[i + 1] for i in range(0, len(vs) - 1, 2)] + (
            [vs[-1]] if len(vs) % 2 else []
        )
    return vs[0]


def _sort_body(x_hbm, out_hbm, buf_a, buf_b, buf_c, hist0, hist1,
               in_sem, out_sem):
    wid = lax.axis_index("s") * NC + lax.axis_index("c")
    iota = lax.iota(jnp.int32, L)
    zeros = iota & np.int32(0)
    mask_d = np.int32(NB - 1)
    base_row = wid * ROWS_PER_W

    def digit(k, shift):
        d = k if shift == 0 else lax.shift_right_logical(k, shift)
        return d & mask_d

    def wait_in():
        pltpu.make_async_copy(x_hbm.at[0], buf_b, in_sem).wait()

    def wait_out():
        pltpu.make_async_copy(x_hbm.at[0], buf_b, out_sem).wait()

    def zero_hist(h):
        def zero_body(c, _):
            for u in range(8):
                h[pl.ds(c * (8 * L) + u * L, L)] = zeros
            return 0

        lax.fori_loop(0, HW // (8 * L), zero_body, 0)

    def sort_row(home, row, hook):
        for p in range(NPASS):
            shift = RB * p
            first = p == 0
            last = p == NPASS - 1
            src = home if p % 2 == 0 else buf_b
            dst = buf_b if p % 2 == 0 else home
            cur = hist0 if p % 2 == 0 else hist1
            nxt = hist1 if p % 2 == 0 else hist0
            cur_views = [cur.at[pl.ds(s * NB, NB)] for s in range(S)]

            if first:
                zero_hist(cur)

                def p1_count(ks):
                    ds_ = [digit(k, 0) for k in ks]
                    ccs = [plsc.scan_count(d) for d in ds_]
                    return ds_, ccs

                def p1_add(ds_, ccs):
                    for s in range(S):
                        occ, last_m = ccs[s]
                        plsc.addupdate_scatter(
                            cur_views[s], [ds_[s]], occ, mask=last_m
                        )

                def p1_body(j, ks, src=src):
                    ds_, ccs = p1_count(ks)
                    nsls = [
                        pl.ds(s * SB + (j + 1) * L, L) for s in range(S)
                    ]
                    nks = [_flip(src[sl]) for sl in nsls]
                    for sl, k in zip(nsls, nks):
                        src[sl] = k
                    p1_add(ds_, ccs)
                    return nks

                ks0 = [_flip(src[pl.ds(s * SB, L)]) for s in range(S)]
                for s in range(S):
                    src[pl.ds(s * SB, L)] = ks0[s]
                ksl = lax.fori_loop(0, J - 1, p1_body, ks0)
                p1_add(*p1_count(ksl))
                hook()

            def p2_body(c, carry, cur=cur, nxt=nxt, last=last):
                vs = [cur[pl.ds(s * NB + c * L, L)] for s in range(S)]
                t = _tree_sum(vs)
                cum = plsc.cumsum(t)
                acc = cum - t + carry
                for s in range(S):
                    cur[pl.ds(s * NB + c * L, L)] = acc
                    if not last:
                        nxt[pl.ds(s * NB + c * L, L)] = zeros
                    acc = acc + vs[s]
                return carry + jnp.sum(t)

            lax.fori_loop(0, NB // L, p2_body, np.int32(0))

            def p3_body(j, ks, src=src, dst=dst, shift=shift, last=last,
                        cur_views=cur_views, nxt=nxt):
                ds_ = [digit(k, shift) for k in ks]
                ccs = [plsc.scan_count(d) for d in ds_]
                jn = (j + 1) & np.int32(J - 1)
                nks = [src[pl.ds(s * SB + jn * L, L)] for s in range(S)]
                rbs = [
                    plsc.load_gather(cur_views[s], [ds_[s]])
                    for s in range(S)
                ]
                qs = []
                for s in range(S):
                    occ, last_m = ccs[s]
                    val = rbs[s] + occ
                    plsc.store_scatter(
                        cur_views[s], [ds_[s]], val, mask=last_m
                    )
                    qs.append(val + np.int32(-1))
                if last:
                    outs = [_unflip(k) for k in ks]
                else:
                    outs = ks
                    nis = [
                        (
                            lax.shift_right_logical(q, LOG_SB - RB)
                            & np.int32(((S - 1) << RB))
                        )
                        | digit(k, shift + RB)
                        for q, k in zip(qs, ks)
                    ]
                    nccs = [plsc.scan_count(ni) for ni in nis]
                for s in range(S):
                    plsc.store_scatter(dst, [qs[s]], outs[s])
                if not last:
                    for s in range(S):
                        nocc, nlast = nccs[s]
                        plsc.addupdate_scatter(
                            nxt, [nis[s]], nocc, mask=nlast
                        )
                return nks

            ks0 = [src[pl.ds(s * SB, L)] for s in range(S)]
            lax.fori_loop(0, J, p3_body, ks0)

    def pair_body(k, _):
        row_a = base_row + 2 * k
        row_b = row_a + 1

        def hook_a(k=k, row_b=row_b):
            @pl.when(k > 0)
            def _():
                wait_out()

            pltpu.async_copy(x_hbm.at[row_b], buf_c, in_sem)

        @pl.when(k > 0)
        def _():
            wait_in()

        sort_row(buf_a, row_a, hook_a)
        pltpu.async_copy(buf_a, out_hbm.at[row_a], out_sem)
        wait_in()

        def hook_b(k=k, row_a=row_a):
            wait_out()

            @pl.when(k == 0)
            def _():
                pltpu.async_copy(x_hbm.at[row_a + 2], buf_a, in_sem)

        sort_row(buf_c, row_b, hook_b)
        pltpu.async_copy(buf_c, out_hbm.at[row_b], out_sem)
        return 0

    pltpu.sync_copy(x_hbm.at[base_row], buf_a)
    lax.fori_loop(0, ROWS_PER_W // 2, pair_body, 0)
    wait_out()


_mesh = plsc.VectorSubcoreMesh(
    core_axis_name="c", subcore_axis_name="s", num_cores=NC, num_subcores=NS
)

_sort = pl.kernel(
    _sort_body,
    out_type=jax.ShapeDtypeStruct((ROWS, N), jnp.int32),
    mesh=_mesh,
    scratch_types=[
        pltpu.VMEM((N,), jnp.int32),
        pltpu.VMEM((N,), jnp.int32),
        pltpu.VMEM((N,), jnp.int32),
        pltpu.VMEM((HW,), jnp.int32),
        pltpu.VMEM((HW,), jnp.int32),
        pltpu.SemaphoreType.DMA,
        pltpu.SemaphoreType.DMA,
    ],
    compiler_params=pltpu.CompilerParams(needs_layout_passes=False),
)


@jax.jit
def kernel(x):
    x_i32 = lax.bitcast_convert_type(x, jnp.int32)
    out = _sort(x_i32)
    return lax.bitcast_convert_type(out, jnp.float32)


# speedup vs baseline: 10.8587x; 1.0048x over previous
import jax
import jax.numpy as jnp
import numpy as np
from jax import lax
from jax.experimental import pallas as pl
from jax.experimental.pallas import tpu as pltpu
from jax.experimental.pallas import tpu_sc as plsc

NC = 2
NS = 16
NW = NC * NS
L = 16

ROWS = 128
N = 32768
V = N // L
S = 8
J = V // S
RB = 8
NB = 1 << RB
HW = S * NB
NPASS = 4
ROWS_PER_W = ROWS // NW
SB = J * L
LOG_SB = 12

_MIN_I32 = np.int32(-2147483648)


def _flip(k):
    m = lax.shift_right_arithmetic(k, 31) | _MIN_I32
    return k ^ m


def _unflip(k):
    m = lax.shift_right_arithmetic(~k, 31) | _MIN_I32
    return k ^ m


def _tree_sum(vs):
    while len(vs) > 1:
        vs = [vs[i] + vs[i + 1] for i in range(0, len(vs) - 1, 2)] + (
            [vs[-1]] if len(vs) % 2 else []
        )
    return vs[0]


def _sort_body(x_hbm, out_hbm, buf_a, buf_b, buf_c, hist0, hist1,
               in_sem, out_sem):
    wid = lax.axis_index("s") * NC + lax.axis_index("c")
    iota = lax.iota(jnp.int32, L)
    zeros = iota & np.int32(0)
    mask_d = np.int32(NB - 1)
    base_row = wid * ROWS_PER_W

    def digit(k, shift):
        d = k if shift == 0 else lax.shift_right_logical(k, shift)
        return d & mask_d

    def wait_in():
        pltpu.make_async_copy(x_hbm.at[0], buf_b, in_sem).wait()

    def wait_out():
        pltpu.make_async_copy(x_hbm.at[0], buf_b, out_sem).wait()

    def zero_hist(h):
        def zero_body(c, _):
            for u in range(8):
                h[pl.ds(c * (8 * L) + u * L, L)] = zeros
            return 0

        lax.fori_loop(0, HW // (8 * L), zero_body, 0)

    def sort_row(home, row, hook):
        for p in range(NPASS):
            shift = RB * p
            first = p == 0
            last = p == NPASS - 1
            src = home if p % 2 == 0 else buf_b
            dst = buf_b if p % 2 == 0 else home
            cur = hist0 if p % 2 == 0 else hist1
            nxt = hist1 if p % 2 == 0 else hist0
            cur_views = [cur.at[pl.ds(s * NB, NB)] for s in range(S)]

            if first:
                zero_hist(cur)

                def p1_count(ks):
                    ds_ = [digit(k, 0) for k in ks]
                    ccs = [plsc.scan_count(d) for d in ds_]
                    return ds_, ccs

                def p1_add(ds_, ccs):
                    for s in range(S):
                        occ, last_m = ccs[s]
                        plsc.addupdate_scatter(
                            cur_views[s], [ds_[s]], occ, mask=last_m
                        )

                def p1_body(j, ks, src=src):
                    ds_, ccs = p1_count(ks)
                    nsls = [
                        pl.ds(s * SB + (j + 1) * L, L) for s in range(S)
                    ]
                    nks = [_flip(src[sl]) for sl in nsls]
                    p1_add(ds_, ccs)
                    for sl, k in zip(nsls, nks):
                        src[sl] = k
                    return nks

                ks0 = [_flip(src[pl.ds(s * SB, L)]) for s in range(S)]
                for s in range(S):
                    src[pl.ds(s * SB, L)] = ks0[s]
                ksl = lax.fori_loop(0, J - 1, p1_body, ks0)
                p1_add(*p1_count(ksl))
                hook()

            def p2_body(c, carry, cur=cur, nxt=nxt, last=last):
                vs = [cur[pl.ds(s * NB + c * L, L)] for s in range(S)]
                t = _tree_sum(vs)
                cum = plsc.cumsum(t)
                acc = cum - t + carry
                for s in range(S):
                    cur[pl.ds(s * NB + c * L, L)] = acc
                    if not last:
                        nxt[pl.ds(s * NB + c * L, L)] = zeros
                    acc = acc + vs[s]
                return carry + jnp.sum(t)

            lax.fori_loop(0, NB // L, p2_body, np.int32(0))

            def p3_body(j, ks, src=src, dst=dst, shift=shift, last=last,
                        cur_views=cur_views, nxt=nxt):
                ds_ = [digit(k, shift) for k in ks]
                ccs = [plsc.scan_count(d) for d in ds_]
                jn = (j + 1) & np.int32(J - 1)
                nks = [src[pl.ds(s * SB + jn * L, L)] for s in range(S)]
                rbs = [
                    plsc.load_gather(cur_views[s], [ds_[s]])
                    for s in range(S)
                ]
                qs = []
                for s in range(S):
                    occ, last_m = ccs[s]
                    val = rbs[s] + occ
                    plsc.store_scatter(
                        cur_views[s], [ds_[s]], val, mask=last_m
                    )
                    qs.append(val + np.int32(-1))
                if last:
                    outs = [_unflip(k) for k in ks]
                else:
                    outs = ks
                    nis = [
                        (
                            lax.shift_right_logical(q, LOG_SB - RB)
                            & np.int32(((S - 1) << RB))
                        )
                        | digit(k, shift + RB)
                        for q, k in zip(qs, ks)
                    ]
                    nccs = [plsc.scan_count(ni) for ni in nis]
                for s in range(S):
                    plsc.store_scatter(dst, [qs[s]], outs[s])
                if not last:
                    for s in range(S):
                        nocc, nlast = nccs[s]
                        plsc.addupdate_scatter(
                            nxt, [nis[s]], nocc, mask=nlast
                        )
                return nks

            ks0 = [src[pl.ds(s * SB, L)] for s in range(S)]
            lax.fori_loop(0, J, p3_body, ks0)

    def pair_body(k, _):
        row_a = base_row + 2 * k
        row_b = row_a + 1

        def hook_a(k=k, row_b=row_b):
            @pl.when(k > 0)
            def _():
                wait_out()

            pltpu.async_copy(x_hbm.at[row_b], buf_c, in_sem)

        @pl.when(k > 0)
        def _():
            wait_in()

        sort_row(buf_a, row_a, hook_a)
        pltpu.async_copy(buf_a, out_hbm.at[row_a], out_sem)
        wait_in()

        def hook_b(k=k, row_a=row_a):
            wait_out()

            @pl.when(k == 0)
            def _():
                pltpu.async_copy(x_hbm.at[row_a + 2], buf_a, in_sem)

        sort_row(buf_c, row_b, hook_b)
        pltpu.async_copy(buf_c, out_hbm.at[row_b], out_sem)
        return 0

    pltpu.sync_copy(x_hbm.at[base_row], buf_a)
    lax.fori_loop(0, ROWS_PER_W // 2, pair_body, 0)
    wait_out()


_mesh = plsc.VectorSubcoreMesh(
    core_axis_name="c", subcore_axis_name="s", num_cores=NC, num_subcores=NS
)

_sort = pl.kernel(
    _sort_body,
    out_type=jax.ShapeDtypeStruct((ROWS, N), jnp.int32),
    mesh=_mesh,
    scratch_types=[
        pltpu.VMEM((N,), jnp.int32),
        pltpu.VMEM((N,), jnp.int32),
        pltpu.VMEM((N,), jnp.int32),
        pltpu.VMEM((HW,), jnp.int32),
        pltpu.VMEM((HW,), jnp.int32),
        pltpu.SemaphoreType.DMA,
        pltpu.SemaphoreType.DMA,
    ],
    compiler_params=pltpu.CompilerParams(needs_layout_passes=False),
)


@jax.jit
def kernel(x):
    x_i32 = lax.bitcast_convert_type(x, jnp.int32)
    out = _sort(x_i32)
    return lax.bitcast_convert_type(out, jnp.float32)
